# Initial kernel scaffold; baseline (speedup 1.0000x reference)
import jax
import jax.numpy as jnp
from jax.experimental import pallas as pl


def kernel(input_ids, word_emb, pos_emb, tok_emb, ln_weight, ln_bias):
    raise NotImplementedError("write your pallas kernel here")


# speedup vs baseline: 1.5001x; 1.5001x over previous
import functools

import jax
import jax.numpy as jnp
from jax import lax
from jax.experimental import pallas as pl
from jax.experimental.pallas import tpu as pltpu
from jax.experimental.pallas import tpu_sc as plsc

B = 8
S = 384
H = 768
NTOK = B * S


@functools.cache
def _make_sc_gather():
    info = plsc.get_sparse_core_info()
    nc, ns = info.num_cores, info.num_subcores
    nw = nc * ns
    per_w = NTOK // nw

    mesh = plsc.VectorSubcoreMesh(core_axis_name="c", subcore_axis_name="s")

    @functools.partial(
        pl.kernel,
        mesh=mesh,
        out_type=jax.ShapeDtypeStruct((NTOK, H), jnp.float32),
        scratch_types=[
            pltpu.VMEM((per_w,), jnp.int32),
            pltpu.VMEM((per_w, H), jnp.float32),
            pltpu.SemaphoreType.DMA,
        ],
    )
    def sc_gather(idx_hbm, table_hbm, out_hbm, idx_v, rows_v, sem):
        wid = lax.axis_index("s") * nc + lax.axis_index("c")
        base = wid * per_w
        pltpu.sync_copy(idx_hbm.at[pl.ds(base, per_w)], idx_v)
        pltpu.async_copy(table_hbm.at[idx_v], rows_v, sem).wait()
        pltpu.sync_copy(rows_v, out_hbm.at[pl.ds(base, per_w)])

    return sc_gather


def _ln_body(g_ref, pos_ref, tok_ref, w_ref, b_ref, out_ref):
    x = g_ref[0] + pos_ref[...] + tok_ref[...]
    mean = jnp.mean(x, axis=1, keepdims=True)
    zm = x - mean
    var = jnp.mean(zm * zm, axis=1, keepdims=True)
    y = zm * lax.rsqrt(var + 1e-5)
    yt = y.T
    out_ref[0, :, 0, :] = yt * w_ref[0, 0][:, None] + b_ref[0, 0][:, None]


def kernel(input_ids, word_emb, pos_emb, tok_emb, ln_weight, ln_bias):
    idx = input_ids.reshape(-1).astype(jnp.int32)
    gathered = _make_sc_gather()(idx, word_emb)
    g3 = gathered.reshape(B, S, H)
    w2 = ln_weight.reshape(B, 1, H)
    b2 = ln_bias.reshape(B, 1, H)
    out = pl.pallas_call(
        _ln_body,
        grid=(B,),
        in_specs=[
            pl.BlockSpec((1, S, H), lambda i: (i, 0, 0)),
            pl.BlockSpec((S, H), lambda i: (0, 0)),
            pl.BlockSpec((S, H), lambda i: (0, 0)),
            pl.BlockSpec((1, 1, H), lambda i: (i, 0, 0)),
            pl.BlockSpec((1, 1, H), lambda i: (i, 0, 0)),
        ],
        out_specs=pl.BlockSpec((1, H, 1, S), lambda i: (i, 0, 0, 0)),
        out_shape=jax.ShapeDtypeStruct((B, H, 1, S), jnp.float32),
    )(g3, pos_emb, tok_emb, w2, b2)
    return out


# speedup vs baseline: 1.5041x; 1.0027x over previous
import functools

import jax
import jax.numpy as jnp
from jax import lax
from jax.experimental import pallas as pl
from jax.experimental.pallas import tpu as pltpu
from jax.experimental.pallas import tpu_sc as plsc

B = 8
S = 384
H = 768
NTOK = B * S
SBLK = 384
NCH = 3
CSZ = 32


@functools.cache
def _make_sc_gather():
    info = plsc.get_sparse_core_info()
    nc, ns = info.num_cores, info.num_subcores
    nw = nc * ns
    per_w = NTOK // nw
    chunks_per_batch_row = S // per_w

    mesh = plsc.VectorSubcoreMesh(core_axis_name="c", subcore_axis_name="s")

    @functools.partial(
        pl.kernel,
        mesh=mesh,
        out_type=jax.ShapeDtypeStruct((NTOK, H), jnp.float32),
        scratch_types=[
            pltpu.VMEM((NCH, CSZ), jnp.int32),
            pltpu.VMEM((per_w, H), jnp.float32),
        ]
        + [pltpu.SemaphoreType.DMA] * (2 * NCH),
    )
    def sc_gather(ids_hbm, table_hbm, out_hbm, idx_v, rows_v, *sems):
        gsems, wsems = sems[:NCH], sems[NCH:]
        wid = lax.axis_index("s") * nc + lax.axis_index("c")
        b = wid // chunks_per_batch_row
        s0 = (wid % chunks_per_batch_row) * per_w
        base = wid * per_w
        for c in range(NCH):
            pltpu.sync_copy(ids_hbm.at[b, pl.ds(s0 + c * CSZ, CSZ)], idx_v.at[c])
        gathers = [
            pltpu.async_copy(
                table_hbm.at[idx_v.at[c]], rows_v.at[pl.ds(c * CSZ, CSZ)], gsems[c]
            )
            for c in range(NCH)
        ]
        writes = []
        for c in range(NCH):
            gathers[c].wait()
            writes.append(
                pltpu.async_copy(
                    rows_v.at[pl.ds(c * CSZ, CSZ)],
                    out_hbm.at[pl.ds(base + c * CSZ, CSZ)],
                    wsems[c],
                )
            )
        for w in writes:
            w.wait()

    return sc_gather


def _ln_body(g_ref, pos_ref, tok_ref, w_ref, b_ref, out_ref):
    x = g_ref[0] + pos_ref[...] + tok_ref[...]
    xt = x.T
    mean = jnp.mean(xt, axis=0, keepdims=True)
    zm = xt - mean
    var = jnp.mean(zm * zm, axis=0, keepdims=True)
    y = zm * lax.rsqrt(var + 1e-5)
    out_ref[0, :, 0, :] = y * w_ref[0, 0][:, None] + b_ref[0, 0][:, None]


def _ln_call(g3, pos_emb, tok_emb, w2, b2, interpret=False):
    return pl.pallas_call(
        _ln_body,
        grid=(S // SBLK, B),
        in_specs=[
            pl.BlockSpec((1, SBLK, H), lambda sb, i: (i, sb, 0)),
            pl.BlockSpec((SBLK, H), lambda sb, i: (sb, 0)),
            pl.BlockSpec((SBLK, H), lambda sb, i: (sb, 0)),
            pl.BlockSpec((1, 1, H), lambda sb, i: (i, 0, 0)),
            pl.BlockSpec((1, 1, H), lambda sb, i: (i, 0, 0)),
        ],
        out_specs=pl.BlockSpec((1, H, 1, SBLK), lambda sb, i: (i, 0, 0, sb)),
        out_shape=jax.ShapeDtypeStruct((B, H, 1, S), jnp.float32),
        interpret=interpret,
    )(g3, pos_emb, tok_emb, w2, b2)


def kernel(input_ids, word_emb, pos_emb, tok_emb, ln_weight, ln_bias):
    ids = input_ids.astype(jnp.int32)
    gathered = _make_sc_gather()(ids, word_emb)
    return _ln_call(
        gathered.reshape(B, S, H),
        pos_emb,
        tok_emb,
        ln_weight.reshape(B, 1, H),
        ln_bias.reshape(B, 1, H),
    )
